# baseline (speedup 1.0000x reference)
import functools

import jax
import jax.numpy as jnp
from jax import lax
from jax.experimental import pallas as pl
from jax.experimental.pallas import tpu as pltpu
from jax.experimental.pallas import tpu_sc as plsc

B = 16384
EMB = 16
SUB = 128 // EMB
NC, NS = 2, 16
NW = NC * NS
B_PER_W = B // NW


def _sc_gather(uid_super, mid_super, user_sup, movie_sup):
    mesh = plsc.VectorSubcoreMesh(core_axis_name="c", subcore_axis_name="s")

    @functools.partial(
        pl.kernel,
        mesh=mesh,
        out_type=[
            jax.ShapeDtypeStruct((B, 128), jnp.float32),
            jax.ShapeDtypeStruct((B, 128), jnp.float32),
        ],
        scratch_types=[
            pltpu.VMEM((B_PER_W,), jnp.int32),
            pltpu.VMEM((B_PER_W, 128), jnp.float32),
            pltpu.SemaphoreType.DMA,
        ],
    )
    def gather_kernel(uid_hbm, mid_hbm, ut_hbm, mt_hbm, ue_hbm, me_hbm,
                      idx_v, rows_v, sem):
        wid = lax.axis_index("s") * NC + lax.axis_index("c")
        base = wid * B_PER_W
        pltpu.sync_copy(uid_hbm.at[pl.ds(base, B_PER_W)], idx_v)
        pltpu.async_copy(ut_hbm.at[idx_v], rows_v, sem).wait()
        pltpu.sync_copy(rows_v, ue_hbm.at[pl.ds(base, B_PER_W)])
        pltpu.sync_copy(mid_hbm.at[pl.ds(base, B_PER_W)], idx_v)
        pltpu.async_copy(mt_hbm.at[idx_v], rows_v, sem).wait()
        pltpu.sync_copy(rows_v, me_hbm.at[pl.ds(base, B_PER_W)])

    return gather_kernel(uid_super, mid_super, user_sup, movie_sup)


def _mlp_body(u_ref, m_ref, usub_ref, msub_ref, w1u_ref, w1m_ref, b1_ref,
              w2t_ref, b2_ref, w3t_ref, b3_ref, o_ref):
    blk = u_ref.shape[0]
    lane_group = lax.broadcasted_iota(jnp.int32, (blk, 128), 1) // EMB
    u_masked = jnp.where(lane_group == usub_ref[...], u_ref[...], 0.0)
    m_masked = jnp.where(lane_group == msub_ref[...], m_ref[...], 0.0)
    x1 = jnp.dot(u_masked, w1u_ref[...], preferred_element_type=jnp.float32)
    x1 += jnp.dot(m_masked, w1m_ref[...], preferred_element_type=jnp.float32)
    h1 = jnp.maximum(x1 + b1_ref[...], 0.0)
    h2 = jnp.maximum(
        jnp.dot(h1, w2t_ref[...], preferred_element_type=jnp.float32) + b2_ref[...],
        0.0,
    )
    o_ref[...] = (
        jnp.dot(h2, w3t_ref[...], preferred_element_type=jnp.float32) + b3_ref[...]
    )


def _tc_mlp(u_super, m_super, usub, msub, W1u8, W1m8, b1, W2t, b2, W3t, b3):
    blk = 4096
    grid = B // blk
    row_block = lambda i: (i, 0)
    full = lambda i: (0, 0)
    return pl.pallas_call(
        _mlp_body,
        grid=(grid,),
        in_specs=[
            pl.BlockSpec((blk, 128), row_block),
            pl.BlockSpec((blk, 128), row_block),
            pl.BlockSpec((blk, 1), row_block),
            pl.BlockSpec((blk, 1), row_block),
            pl.BlockSpec((128, 128), full),
            pl.BlockSpec((128, 128), full),
            pl.BlockSpec((1, 128), full),
            pl.BlockSpec((128, 64), full),
            pl.BlockSpec((1, 64), full),
            pl.BlockSpec((64, 1), full),
            pl.BlockSpec((1, 1), full),
        ],
        out_specs=pl.BlockSpec((blk, 1), row_block),
        out_shape=jax.ShapeDtypeStruct((B, 1), jnp.float32),
    )(u_super, m_super, usub, msub, W1u8, W1m8, b1, W2t, b2, W3t, b3)


@jax.jit
def kernel(userId, movieId, user_table, movie_table, W1, b1, W2, b2, W3, b3):
    uid_super = userId // SUB
    usub = (userId % SUB)[:, None]
    mid_super = movieId // SUB
    msub = (movieId % SUB)[:, None]
    user_sup = user_table.reshape(-1, 128)
    movie_sup = movie_table.reshape(-1, 128)
    u_super, m_super = _sc_gather(uid_super, mid_super, user_sup, movie_sup)
    W1u8 = jnp.tile(W1[:, :EMB].T, (SUB, 1))
    W1m8 = jnp.tile(W1[:, EMB:].T, (SUB, 1))
    W2t = W2.T
    W3t = W3.T
    return _tc_mlp(u_super, m_super, usub, msub, W1u8, W1m8, b1[None, :],
                   W2t, b2[None, :], W3t, b3[None, :])


# speedup vs baseline: 1.3074x; 1.3074x over previous
import functools

import jax
import jax.numpy as jnp
from jax import lax
from jax.experimental import pallas as pl
from jax.experimental.pallas import tpu as pltpu
from jax.experimental.pallas import tpu_sc as plsc

B = 16384
EMB = 16
SUB = 128 // EMB
NC, NS = 2, 16
NW = NC * NS
B_PER_W = B // NW
CH = 8192


def _repack_body(in_ref, o_ref):
    x = in_ref[...]
    y3 = x.T.reshape(CH // SUB, SUB, EMB)
    o_ref[...] = jnp.concatenate([y3[:, ul, :] for ul in range(SUB)], axis=1)


def _repack(table_t):
    rows = table_t.shape[1]
    grid = (rows + CH - 1) // CH
    return pl.pallas_call(
        _repack_body,
        grid=(grid,),
        in_specs=[pl.BlockSpec((EMB, CH), lambda i: (0, i))],
        out_specs=pl.BlockSpec((CH // SUB, 128), lambda i: (i, 0)),
        out_shape=jax.ShapeDtypeStruct((grid * CH // SUB, 128), jnp.float32),
        compiler_params=pltpu.CompilerParams(
            dimension_semantics=("parallel",)),
    )(table_t)


def _sc_gather(uid_super, mid_super, user_sup, movie_sup):
    mesh = plsc.VectorSubcoreMesh(core_axis_name="c", subcore_axis_name="s")

    @functools.partial(
        pl.kernel,
        mesh=mesh,
        out_type=[
            jax.ShapeDtypeStruct((B, 128), jnp.float32),
            jax.ShapeDtypeStruct((B, 128), jnp.float32),
        ],
        scratch_types=[
            pltpu.VMEM((B_PER_W,), jnp.int32),
            pltpu.VMEM((B_PER_W, 128), jnp.float32),
            pltpu.SemaphoreType.DMA,
        ],
    )
    def gather_kernel(uid_hbm, mid_hbm, ut_hbm, mt_hbm, ue_hbm, me_hbm,
                      idx_v, rows_v, sem):
        wid = lax.axis_index("s") * NC + lax.axis_index("c")
        base = wid * B_PER_W
        pltpu.sync_copy(uid_hbm.at[pl.ds(base, B_PER_W)], idx_v)
        pltpu.async_copy(ut_hbm.at[idx_v], rows_v, sem).wait()
        pltpu.sync_copy(rows_v, ue_hbm.at[pl.ds(base, B_PER_W)])
        pltpu.sync_copy(mid_hbm.at[pl.ds(base, B_PER_W)], idx_v)
        pltpu.async_copy(mt_hbm.at[idx_v], rows_v, sem).wait()
        pltpu.sync_copy(rows_v, me_hbm.at[pl.ds(base, B_PER_W)])

    return gather_kernel(uid_super, mid_super, user_sup, movie_sup)


def _mlp_body(u_ref, m_ref, usub_ref, msub_ref, w1u_ref, w1m_ref, b1_ref,
              w2t_ref, b2_ref, w3t_ref, b3_ref, o_ref):
    blk = u_ref.shape[0]
    lane_group = lax.broadcasted_iota(jnp.int32, (blk, 128), 1) // EMB
    u_masked = jnp.where(lane_group == usub_ref[...], u_ref[...], 0.0)
    m_masked = jnp.where(lane_group == msub_ref[...], m_ref[...], 0.0)
    x1 = jnp.dot(u_masked, w1u_ref[...], preferred_element_type=jnp.float32)
    x1 += jnp.dot(m_masked, w1m_ref[...], preferred_element_type=jnp.float32)
    h1 = jnp.maximum(x1 + b1_ref[...], 0.0)
    h2 = jnp.maximum(
        jnp.dot(h1, w2t_ref[...], preferred_element_type=jnp.float32) + b2_ref[...],
        0.0,
    )
    o_ref[...] = (
        jnp.dot(h2, w3t_ref[...], preferred_element_type=jnp.float32) + b3_ref[...]
    )


def _tc_mlp(u_super, m_super, usub, msub, W1u8, W1m8, b1, W2t, b2, W3t, b3):
    blk = 4096
    grid = B // blk
    row_block = lambda i: (i, 0)
    full = lambda i: (0, 0)
    return pl.pallas_call(
        _mlp_body,
        grid=(grid,),
        in_specs=[
            pl.BlockSpec((blk, 128), row_block),
            pl.BlockSpec((blk, 128), row_block),
            pl.BlockSpec((blk, 1), row_block),
            pl.BlockSpec((blk, 1), row_block),
            pl.BlockSpec((128, 128), full),
            pl.BlockSpec((128, 128), full),
            pl.BlockSpec((1, 128), full),
            pl.BlockSpec((128, 64), full),
            pl.BlockSpec((1, 64), full),
            pl.BlockSpec((64, 1), full),
            pl.BlockSpec((1, 1), full),
        ],
        out_specs=pl.BlockSpec((blk, 1), row_block),
        out_shape=jax.ShapeDtypeStruct((B, 1), jnp.float32),
        compiler_params=pltpu.CompilerParams(
            dimension_semantics=("parallel",)),
    )(u_super, m_super, usub, msub, W1u8, W1m8, b1, W2t, b2, W3t, b3)


@jax.jit
def kernel(userId, movieId, user_table, movie_table, W1, b1, W2, b2, W3, b3):
    uid_super = userId // SUB
    usub = (userId % SUB)[:, None]
    mid_super = movieId // SUB
    msub = (movieId % SUB)[:, None]
    user_sup = _repack(user_table.T)
    movie_sup = _repack(movie_table.T)
    u_super, m_super = _sc_gather(uid_super, mid_super, user_sup, movie_sup)
    W1u8 = jnp.tile(W1[:, :EMB].T, (SUB, 1))
    W1m8 = jnp.tile(W1[:, EMB:].T, (SUB, 1))
    W2t = W2.T
    W3t = W3.T
    return _tc_mlp(u_super, m_super, usub, msub, W1u8, W1m8, b1[None, :],
                   W2t, b2[None, :], W3t, b3[None, :])


# speedup vs baseline: 2.1508x; 1.6451x over previous
import functools

import jax
import jax.numpy as jnp
from jax import lax
from jax.experimental import pallas as pl
from jax.experimental.pallas import tpu as pltpu
from jax.experimental.pallas import tpu_sc as plsc

B = 16384
EMB = 16
SUB = 128 // EMB
NC, NS = 2, 16
NW = NC * NS
B_PER_W = B // NW
CH = 8192


S8 = CH // SUB


def _repack_body(rows, in_ref, eye_ref, o_ref):
    x = in_ref[...]
    valid = rows - pl.program_id(0) * CH
    lane = lax.broadcasted_iota(jnp.int32, (EMB, CH), 1)
    x = jnp.where(lane < valid, x, 0.0)
    acc = jnp.zeros((S8, 128), jnp.float32)
    for ul in range(SUB):
        acc += lax.dot_general(
            x[:, ul * S8:(ul + 1) * S8],
            eye_ref[EMB * ul:EMB * (ul + 1), :],
            (((0,), (0,)), ((), ())),
            preferred_element_type=jnp.float32,
        )
    o_ref[...] = acc


def _repack(table_t, eye128):
    rows = table_t.shape[1]
    grid = (rows + CH - 1) // CH
    return pl.pallas_call(
        functools.partial(_repack_body, rows),
        grid=(grid,),
        in_specs=[
            pl.BlockSpec((EMB, CH), lambda i: (0, i)),
            pl.BlockSpec((128, 128), lambda i: (0, 0)),
        ],
        out_specs=pl.BlockSpec((S8, 128), lambda i: (i, 0)),
        out_shape=jax.ShapeDtypeStruct((grid * S8, 128), jnp.float32),
        compiler_params=pltpu.CompilerParams(
            dimension_semantics=("parallel",)),
    )(table_t, eye128)


def _sc_gather(uid_super, mid_super, user_sup, movie_sup):
    mesh = plsc.VectorSubcoreMesh(core_axis_name="c", subcore_axis_name="s")

    @functools.partial(
        pl.kernel,
        mesh=mesh,
        out_type=[
            jax.ShapeDtypeStruct((B, 128), jnp.float32),
            jax.ShapeDtypeStruct((B, 128), jnp.float32),
        ],
        scratch_types=[
            pltpu.VMEM((B_PER_W,), jnp.int32),
            pltpu.VMEM((B_PER_W, 128), jnp.float32),
            pltpu.SemaphoreType.DMA,
        ],
    )
    def gather_kernel(uid_hbm, mid_hbm, ut_hbm, mt_hbm, ue_hbm, me_hbm,
                      idx_v, rows_v, sem):
        wid = lax.axis_index("s") * NC + lax.axis_index("c")
        base = wid * B_PER_W
        pltpu.sync_copy(uid_hbm.at[pl.ds(base, B_PER_W)], idx_v)
        pltpu.async_copy(ut_hbm.at[idx_v], rows_v, sem).wait()
        pltpu.sync_copy(rows_v, ue_hbm.at[pl.ds(base, B_PER_W)])
        pltpu.sync_copy(mid_hbm.at[pl.ds(base, B_PER_W)], idx_v)
        pltpu.async_copy(mt_hbm.at[idx_v], rows_v, sem).wait()
        pltpu.sync_copy(rows_v, me_hbm.at[pl.ds(base, B_PER_W)])

    return gather_kernel(uid_super, mid_super, user_sup, movie_sup)


def _mlp_body(u_ref, m_ref, usub_ref, msub_ref, w1u_ref, w1m_ref, b1_ref,
              w2t_ref, b2_ref, w3t_ref, b3_ref, o_ref):
    blk = u_ref.shape[0]
    lane_group = lax.broadcasted_iota(jnp.int32, (blk, 128), 1) // EMB
    u_masked = jnp.where(lane_group == usub_ref[...], u_ref[...], 0.0)
    m_masked = jnp.where(lane_group == msub_ref[...], m_ref[...], 0.0)
    x1 = jnp.dot(u_masked, w1u_ref[...], preferred_element_type=jnp.float32)
    x1 += jnp.dot(m_masked, w1m_ref[...], preferred_element_type=jnp.float32)
    h1 = jnp.maximum(x1 + b1_ref[...], 0.0)
    h2 = jnp.maximum(
        jnp.dot(h1, w2t_ref[...], preferred_element_type=jnp.float32) + b2_ref[...],
        0.0,
    )
    o_ref[...] = (
        jnp.dot(h2, w3t_ref[...], preferred_element_type=jnp.float32) + b3_ref[...]
    )


def _tc_mlp(u_super, m_super, usub, msub, W1u8, W1m8, b1, W2t, b2, W3t, b3):
    blk = 4096
    grid = B // blk
    row_block = lambda i: (i, 0)
    full = lambda i: (0, 0)
    return pl.pallas_call(
        _mlp_body,
        grid=(grid,),
        in_specs=[
            pl.BlockSpec((blk, 128), row_block),
            pl.BlockSpec((blk, 128), row_block),
            pl.BlockSpec((blk, 1), row_block),
            pl.BlockSpec((blk, 1), row_block),
            pl.BlockSpec((128, 128), full),
            pl.BlockSpec((128, 128), full),
            pl.BlockSpec((1, 128), full),
            pl.BlockSpec((128, 64), full),
            pl.BlockSpec((1, 64), full),
            pl.BlockSpec((64, 1), full),
            pl.BlockSpec((1, 1), full),
        ],
        out_specs=pl.BlockSpec((blk, 1), row_block),
        out_shape=jax.ShapeDtypeStruct((B, 1), jnp.float32),
        compiler_params=pltpu.CompilerParams(
            dimension_semantics=("parallel",)),
    )(u_super, m_super, usub, msub, W1u8, W1m8, b1, W2t, b2, W3t, b3)


@jax.jit
def kernel(userId, movieId, user_table, movie_table, W1, b1, W2, b2, W3, b3):
    uid_super = (userId // CH) * S8 + (userId % S8)
    usub = ((userId % CH) // S8)[:, None]
    mid_super = (movieId // CH) * S8 + (movieId % S8)
    msub = ((movieId % CH) // S8)[:, None]
    eye128 = jnp.eye(128, dtype=jnp.float32)
    user_sup = _repack(user_table.T, eye128)
    movie_sup = _repack(movie_table.T, eye128)
    u_super, m_super = _sc_gather(uid_super, mid_super, user_sup, movie_sup)
    W1u8 = jnp.tile(W1[:, :EMB].T, (SUB, 1))
    W1m8 = jnp.tile(W1[:, EMB:].T, (SUB, 1))
    W2t = W2.T
    W3t = W3.T
    return _tc_mlp(u_super, m_super, usub, msub, W1u8, W1m8, b1[None, :],
                   W2t, b2[None, :], W3t, b3[None, :])


# speedup vs baseline: 2.4514x; 1.1398x over previous
import functools

import jax
import jax.numpy as jnp
from jax import lax
from jax.experimental import pallas as pl
from jax.experimental.pallas import tpu as pltpu
from jax.experimental.pallas import tpu_sc as plsc

B = 16384
EMB = 16
SUB = 128 // EMB
NC, NS = 2, 16
NW = NC * NS
B_PER_W = B // NW
CH = 8192


S8 = CH // SUB


def _repack_body(rows, in_ref, eye_ref, o_ref):
    x = in_ref[...]
    valid = rows - pl.program_id(0) * CH
    lane = lax.broadcasted_iota(jnp.int32, (EMB, CH), 1)
    x = jnp.where(lane < valid, x, 0.0)
    acc = jnp.zeros((128, S8), jnp.float32)
    for ul in range(SUB):
        acc += lax.dot_general(
            eye_ref[:, EMB * ul:EMB * (ul + 1)],
            x[:, ul * S8:(ul + 1) * S8],
            (((1,), (0,)), ((), ())),
            preferred_element_type=jnp.float32,
        )
    o_ref[...] = acc.T


def _repack(table_t, eye128):
    rows = table_t.shape[1]
    grid = (rows + CH - 1) // CH
    return pl.pallas_call(
        functools.partial(_repack_body, rows),
        grid=(grid,),
        in_specs=[
            pl.BlockSpec((EMB, CH), lambda i: (0, i)),
            pl.BlockSpec((128, 128), lambda i: (0, 0)),
        ],
        out_specs=pl.BlockSpec((S8, 128), lambda i: (i, 0)),
        out_shape=jax.ShapeDtypeStruct((grid * S8, 128), jnp.float32),
        compiler_params=pltpu.CompilerParams(
            dimension_semantics=("parallel",),
            fuse_transposed_lhs_in_matmul=True),
    )(table_t, eye128)


def _sc_gather(uid_super, mid_super, user_sup, movie_sup):
    mesh = plsc.VectorSubcoreMesh(core_axis_name="c", subcore_axis_name="s")

    @functools.partial(
        pl.kernel,
        mesh=mesh,
        out_type=[
            jax.ShapeDtypeStruct((B, 128), jnp.float32),
            jax.ShapeDtypeStruct((B, 128), jnp.float32),
        ],
        scratch_types=[
            pltpu.VMEM((B_PER_W,), jnp.int32),
            pltpu.VMEM((B_PER_W, 128), jnp.float32),
            pltpu.SemaphoreType.DMA,
        ],
    )
    def gather_kernel(uid_hbm, mid_hbm, ut_hbm, mt_hbm, ue_hbm, me_hbm,
                      idx_v, rows_v, sem):
        wid = lax.axis_index("s") * NC + lax.axis_index("c")
        base = wid * B_PER_W
        pltpu.sync_copy(uid_hbm.at[pl.ds(base, B_PER_W)], idx_v)
        pltpu.async_copy(ut_hbm.at[idx_v], rows_v, sem).wait()
        pltpu.sync_copy(rows_v, ue_hbm.at[pl.ds(base, B_PER_W)])
        pltpu.sync_copy(mid_hbm.at[pl.ds(base, B_PER_W)], idx_v)
        pltpu.async_copy(mt_hbm.at[idx_v], rows_v, sem).wait()
        pltpu.sync_copy(rows_v, me_hbm.at[pl.ds(base, B_PER_W)])

    return gather_kernel(uid_super, mid_super, user_sup, movie_sup)


def _mlp_body(u_ref, m_ref, usub_ref, msub_ref, w1u_ref, w1m_ref, b1_ref,
              w2t_ref, b2_ref, w3t_ref, b3_ref, o_ref):
    blk = u_ref.shape[0]
    lane_group = lax.broadcasted_iota(jnp.int32, (blk, 128), 1) // EMB
    u_masked = jnp.where(lane_group == usub_ref[...], u_ref[...], 0.0)
    m_masked = jnp.where(lane_group == msub_ref[...], m_ref[...], 0.0)
    x1 = jnp.dot(u_masked, w1u_ref[...], preferred_element_type=jnp.float32)
    x1 += jnp.dot(m_masked, w1m_ref[...], preferred_element_type=jnp.float32)
    h1 = jnp.maximum(x1 + b1_ref[...], 0.0)
    h2 = jnp.maximum(
        jnp.dot(h1, w2t_ref[...], preferred_element_type=jnp.float32) + b2_ref[...],
        0.0,
    )
    o_ref[...] = (
        jnp.dot(h2, w3t_ref[...], preferred_element_type=jnp.float32) + b3_ref[...]
    )


def _tc_mlp(u_super, m_super, usub, msub, W1u8, W1m8, b1, W2t, b2, W3t, b3):
    blk = 4096
    grid = B // blk
    row_block = lambda i: (i, 0)
    full = lambda i: (0, 0)
    return pl.pallas_call(
        _mlp_body,
        grid=(grid,),
        in_specs=[
            pl.BlockSpec((blk, 128), row_block),
            pl.BlockSpec((blk, 128), row_block),
            pl.BlockSpec((blk, 1), row_block),
            pl.BlockSpec((blk, 1), row_block),
            pl.BlockSpec((128, 128), full),
            pl.BlockSpec((128, 128), full),
            pl.BlockSpec((1, 128), full),
            pl.BlockSpec((128, 64), full),
            pl.BlockSpec((1, 64), full),
            pl.BlockSpec((64, 1), full),
            pl.BlockSpec((1, 1), full),
        ],
        out_specs=pl.BlockSpec((blk, 1), row_block),
        out_shape=jax.ShapeDtypeStruct((B, 1), jnp.float32),
        compiler_params=pltpu.CompilerParams(
            dimension_semantics=("parallel",)),
    )(u_super, m_super, usub, msub, W1u8, W1m8, b1, W2t, b2, W3t, b3)


@jax.jit
def kernel(userId, movieId, user_table, movie_table, W1, b1, W2, b2, W3, b3):
    uid_super = (userId // CH) * S8 + (userId % S8)
    usub = ((userId % CH) // S8)[:, None]
    mid_super = (movieId // CH) * S8 + (movieId % S8)
    msub = ((movieId % CH) // S8)[:, None]
    eye128 = jnp.eye(128, dtype=jnp.float32)
    user_sup = _repack(user_table.T, eye128)
    movie_sup = _repack(movie_table.T, eye128)
    u_super, m_super = _sc_gather(uid_super, mid_super, user_sup, movie_sup)
    W1u8 = jnp.tile(W1[:, :EMB].T, (SUB, 1))
    W1m8 = jnp.tile(W1[:, EMB:].T, (SUB, 1))
    W2t = W2.T
    W3t = W3.T
    return _tc_mlp(u_super, m_super, usub, msub, W1u8, W1m8, b1[None, :],
                   W2t, b2[None, :], W3t, b3[None, :])
